# Initial kernel scaffold; baseline (speedup 1.0000x reference)
import jax
import jax.numpy as jnp
from jax.experimental import pallas as pl


def kernel(input, G, gate_proj, up_proj, down_proj, lora_A, lora_B):
    raise NotImplementedError("write your pallas kernel here")


# speedup vs baseline: 6.6460x; 6.6460x over previous
import functools
import jax
import jax.numpy as jnp
from jax.experimental import pallas as pl
from jax.experimental.pallas import tpu as pltpu

H = 1024
E = 8
FFH = H // E
LORA_R = 16
LORA_SCALE = 2.0
HID = E * FFH
LR = E * LORA_R
TB = 512


def _fused_kernel(x_ref, g_ref, win_ref, wout_ref, o_ref):
    xb = x_ref[...]

    logits = jnp.dot(xb, g_ref[...], preferred_element_type=jnp.float32)
    col = jax.lax.broadcasted_iota(jnp.int32, logits.shape, 1)
    logits = jnp.where(col < E, logits, -1e30)
    m1 = jnp.max(logits, axis=-1, keepdims=True)
    idx1 = jnp.min(jnp.where(logits == m1, col, E), axis=-1, keepdims=True)
    l2 = jnp.where(col == idx1, -1e30, logits)
    m2 = jnp.max(l2, axis=-1, keepdims=True)
    idx2 = jnp.min(jnp.where(l2 == m2, col, E), axis=-1, keepdims=True)
    t = jnp.exp(m2 - m1)
    w1 = 1.0 / (1.0 + t)
    w2 = t / (1.0 + t)

    xb16 = xb.astype(jnp.bfloat16)
    acts = jnp.dot(xb16, win_ref[...], preferred_element_type=jnp.float32)
    a = acts[:, :HID]
    u = acts[:, HID:2 * HID]
    l = acts[:, 2 * HID:]
    h = (a / (1.0 + jnp.exp(-a))) * u

    hcol = jax.lax.broadcasted_iota(jnp.int32, h.shape, 1) // FFH
    wh = jnp.where(hcol == idx1, w1, 0.0) + jnp.where(hcol == idx2, w2, 0.0)
    lcol = jax.lax.broadcasted_iota(jnp.int32, l.shape, 1) // LORA_R
    wl = jnp.where(lcol == idx1, w1, 0.0) + jnp.where(lcol == idx2, w2, 0.0)
    hw = jnp.concatenate(
        [(h * wh).astype(jnp.bfloat16), (l * (LORA_SCALE * wl)).astype(jnp.bfloat16)],
        axis=1)

    o_ref[...] = jnp.dot(hw, wout_ref[...], preferred_element_type=jnp.float32)


@functools.partial(jax.jit, static_argnames=("interpret",))
def _run(xt, g_pad, w_in, w_out, interpret=False):
    n = xt.shape[0]
    return pl.pallas_call(
        _fused_kernel,
        grid=(n // TB,),
        in_specs=[
            pl.BlockSpec((TB, H), lambda i: (i, 0)),
            pl.BlockSpec((H, 128), lambda i: (0, 0)),
            pl.BlockSpec((H, 2 * HID + LR), lambda i: (0, 0)),
            pl.BlockSpec((HID + LR, H), lambda i: (0, 0)),
        ],
        out_specs=pl.BlockSpec((TB, H), lambda i: (i, 0)),
        out_shape=jax.ShapeDtypeStruct((n, H), jnp.float32),
        compiler_params=pltpu.CompilerParams(
            dimension_semantics=("arbitrary",)),
        interpret=interpret,
    )(xt, g_pad, w_in, w_out)


def kernel(input, G, gate_proj, up_proj, down_proj, lora_A, lora_B,
           interpret=False):
    b, s, h = input.shape
    xt = input.reshape(-1, h)
    g_pad = jnp.pad(G, ((0, 0), (0, 128 - E)))
    gate_t = gate_proj.reshape(HID, H).T
    up_t = up_proj.reshape(HID, H).T
    la_t = lora_A.reshape(LR, H).T
    w_in = jnp.concatenate([gate_t, up_t, la_t], axis=1).astype(jnp.bfloat16)
    down_s = down_proj.transpose(0, 2, 1).reshape(HID, H)
    lb_s = lora_B.transpose(0, 2, 1).reshape(LR, H)
    w_out = jnp.concatenate([down_s, lb_s], axis=0).astype(jnp.bfloat16)
    out = _run(xt, g_pad, w_in, w_out, interpret=interpret)
    return out.reshape(b, s, h)
